# baseline (speedup 1.0000x reference)
import functools

import jax
import jax.numpy as jnp
from jax import lax
from jax.experimental import pallas as pl
from jax.experimental.pallas import tpu as pltpu
from jax.experimental.pallas import tpu_sc as plsc

N_NODES = 10000
D = 128
N_EDGES = 320000

NC = 2
NS = 16
NW = NC * NS
CHUNK = 128
EPT = 10112
NCHUNK = EPT // CHUNK
E_PAD = NW * EPT
NPAD = 10240
ZROWS = NPAD // NS


def _make_spmm():
  mesh = plsc.VectorSubcoreMesh(core_axis_name="c", subcore_axis_name="s")

  @functools.partial(
      pl.kernel,
      mesh=mesh,
      out_type=jax.ShapeDtypeStruct((NC * NPAD, D), jnp.float32),
      scratch_types=[
          pltpu.VMEM((NCHUNK, CHUNK), jnp.int32),
          pltpu.VMEM((NCHUNK, CHUNK), jnp.int32),
          pltpu.VMEM((CHUNK * 16,), jnp.float32),
          pltpu.VMEM((CHUNK, D), jnp.float32),
          pltpu.VMEM_SHARED((NPAD, D), jnp.float32),
          pltpu.SemaphoreType.DMA,
      ],
  )
  def spmm(x_hbm, col_hbm, row_hbm, val_hbm, out_hbm,
           col_v, row_v, val_v, rows_v, acc_sh, sem):
    cid = lax.axis_index("c")
    sid = lax.axis_index("s")
    tid = cid * NS + sid

    pltpu.sync_copy(col_hbm.at[tid], col_v)
    pltpu.sync_copy(row_hbm.at[tid], row_v)

    def zero_row(i, carry):
      for f in range(8):
        rows_v[i, pl.ds(f * 16, 16)] = jnp.zeros((16,), jnp.float32)
      return carry
    lax.fori_loop(0, CHUNK, zero_row, 0)
    for k in range(ZROWS // CHUNK):
      pltpu.sync_copy(rows_v, acc_sh.at[pl.ds(sid * ZROWS + k * CHUNK, CHUNK)])
    plsc.subcore_barrier()

    def chunk_body(j, carry):
      pltpu.sync_copy(val_hbm.at[tid * NCHUNK + j], val_v)
      pltpu.async_copy(x_hbm.at[col_v.at[j]], rows_v, sem).wait()

      def edge_body(e, c2):
        vb = val_v[pl.ds(e * 16, 16)]
        for f in range(8):
          sl = pl.ds(f * 16, 16)
          rows_v[e, sl] = rows_v[e, sl] * vb
        return c2
      lax.fori_loop(0, CHUNK, edge_body, 0)

      pltpu.sync_copy(rows_v, acc_sh.at[row_v.at[j]], add=True)
      return carry
    lax.fori_loop(0, NCHUNK, chunk_body, 0)

    plsc.subcore_barrier()
    pltpu.sync_copy(acc_sh.at[pl.ds(sid * ZROWS, ZROWS)],
                    out_hbm.at[pl.ds(cid * NPAD + sid * ZROWS, ZROWS)])

  return spmm


_spmm = _make_spmm()


_BLK = 1000


def _mm_kernel(x_ref, w_ref, o_ref):
  o_ref[...] = jnp.dot(x_ref[...], w_ref[...],
                       preferred_element_type=jnp.float32)


def _tc_matmul(x, w):
  return pl.pallas_call(
      _mm_kernel,
      grid=(N_NODES // _BLK,),
      in_specs=[
          pl.BlockSpec((_BLK, D), lambda i: (i, 0)),
          pl.BlockSpec((D, D), lambda i: (0, 0)),
      ],
      out_specs=pl.BlockSpec((_BLK, D), lambda i: (i, 0)),
      out_shape=jax.ShapeDtypeStruct((N_NODES, D), jnp.float32),
  )(x, w)


def _fuse_mm_kernel(p_ref, b_ref, w_ref, o_ref):
  h = jnp.maximum(p_ref[0] + p_ref[1] + b_ref[...], 0.0)
  o_ref[...] = jnp.dot(h, w_ref[...], preferred_element_type=jnp.float32)


def _tc_combine_relu_matmul(p, b, w):
  return pl.pallas_call(
      _fuse_mm_kernel,
      grid=(N_NODES // _BLK,),
      in_specs=[
          pl.BlockSpec((2, _BLK, D), lambda i: (0, i, 0)),
          pl.BlockSpec((1, D), lambda i: (0, 0)),
          pl.BlockSpec((D, D), lambda i: (0, 0)),
      ],
      out_specs=pl.BlockSpec((_BLK, D), lambda i: (i, 0)),
      out_shape=jax.ShapeDtypeStruct((N_NODES, D), jnp.float32),
  )(p, b.reshape(1, D), w)


def _fuse_relu_kernel(p_ref, b_ref, o_ref):
  o_ref[...] = jnp.maximum(p_ref[0] + p_ref[1] + b_ref[...], 0.0)


def _tc_combine_relu(p, b):
  return pl.pallas_call(
      _fuse_relu_kernel,
      grid=(N_NODES // _BLK,),
      in_specs=[
          pl.BlockSpec((2, _BLK, D), lambda i: (0, i, 0)),
          pl.BlockSpec((1, D), lambda i: (0, 0)),
      ],
      out_specs=pl.BlockSpec((_BLK, D), lambda i: (i, 0)),
      out_shape=jax.ShapeDtypeStruct((N_NODES, D), jnp.float32),
  )(p, b.reshape(1, D))


def _prep_edges(edge_index, edge_values):
  row = edge_index[0].astype(jnp.int32)
  col = edge_index[1].astype(jnp.int32)
  val = edge_values.astype(jnp.float32)
  pad = E_PAD - N_EDGES
  row = jnp.pad(row, (0, pad)).reshape(NW, NCHUNK, CHUNK)
  col = jnp.pad(col, (0, pad)).reshape(NW, NCHUNK, CHUNK)
  val = jnp.repeat(jnp.pad(val, (0, pad)), 16,
                   total_repeat_length=E_PAD * 16)
  val = val.reshape(NW * NCHUNK, CHUNK * 16)
  return row, col, val


@jax.jit
def _gcn(H, edge_index, edge_values, W0, b0, W1, b1):
  row, col, val = _prep_edges(edge_index, edge_values)
  x0 = _tc_matmul(H, W0)
  p0 = _spmm(x0, col, row, val).reshape(NC, NPAD, D)
  x1 = _tc_combine_relu_matmul(p0, b0, W1)
  p1 = _spmm(x1, col, row, val).reshape(NC, NPAD, D)
  return _tc_combine_relu(p1, b1)


def kernel(H, edge_index, edge_values, W0, b0, W1, b1):
  return _gcn(H, edge_index, edge_values, W0, b0, W1, b1)


# speedup vs baseline: 48.0069x; 48.0069x over previous
import functools

import jax
import jax.numpy as jnp
from jax import lax
from jax.experimental import pallas as pl
from jax.experimental.pallas import tpu as pltpu
from jax.experimental.pallas import tpu_sc as plsc

N_NODES = 10000
D = 128
N_EDGES = 320000

NC = 2
NS = 16
NW = NC * NS
CHUNK = 128
EPT = 10112
NCHUNK = EPT // CHUNK
E_PAD = NW * EPT
NPAD = 10240
ZROWS = NPAD // NS


def _make_spmm():
  mesh = plsc.VectorSubcoreMesh(core_axis_name="c", subcore_axis_name="s")

  @functools.partial(
      pl.kernel,
      mesh=mesh,
      out_type=jax.ShapeDtypeStruct((NC * NPAD, D), jnp.float32),
      scratch_types=[
          pltpu.VMEM((NCHUNK, CHUNK), jnp.int32),
          pltpu.VMEM((NCHUNK, CHUNK), jnp.int32),
          pltpu.VMEM((CHUNK * 16,), jnp.float32),
          pltpu.VMEM((CHUNK, D), jnp.float32),
          pltpu.VMEM_SHARED((NPAD, D), jnp.float32),
          pltpu.SemaphoreType.DMA,
      ],
  )
  def spmm(x_hbm, col_hbm, row_hbm, val_hbm, out_hbm,
           col_v, row_v, val_v, rows_v, acc_sh, sem):
    cid = lax.axis_index("c")
    sid = lax.axis_index("s")
    tid = cid * NS + sid

    pltpu.sync_copy(col_hbm.at[tid], col_v)
    pltpu.sync_copy(row_hbm.at[tid], row_v)

    def zero_row(i, carry):
      for f in range(8):
        rows_v[i, pl.ds(f * 16, 16)] = jnp.zeros((16,), jnp.float32)
      return carry
    lax.fori_loop(0, CHUNK, zero_row, 0)
    for k in range(ZROWS // CHUNK):
      pltpu.sync_copy(rows_v, acc_sh.at[pl.ds(sid * ZROWS + k * CHUNK, CHUNK)])
    plsc.subcore_barrier()

    def chunk_body(j, carry):
      pltpu.sync_copy(val_hbm.at[tid * NCHUNK + j], val_v)
      pltpu.async_copy(x_hbm.at[col_v.at[j]], rows_v, sem).wait()

      def edge_body(e, c2):
        vb = val_v[pl.ds(e * 16, 16)]
        for f in range(8):
          sl = pl.ds(f * 16, 16)
          rows_v[e, sl] = rows_v[e, sl] * vb
        return c2
      lax.fori_loop(0, CHUNK, edge_body, 0)

      pltpu.sync_copy(rows_v, acc_sh.at[row_v.at[j]], add=True)
      return carry
    lax.fori_loop(0, NCHUNK, chunk_body, 0)

    plsc.subcore_barrier()
    pltpu.sync_copy(acc_sh.at[pl.ds(sid * ZROWS, ZROWS)],
                    out_hbm.at[pl.ds(cid * NPAD + sid * ZROWS, ZROWS)])

  return spmm


_spmm = _make_spmm()


_BLK = 1000


def _mm_kernel(x_ref, w_ref, o_ref):
  o_ref[...] = jnp.dot(x_ref[...], w_ref[...],
                       preferred_element_type=jnp.float32)


def _tc_matmul(x, w):
  return pl.pallas_call(
      _mm_kernel,
      grid=(N_NODES // _BLK,),
      in_specs=[
          pl.BlockSpec((_BLK, D), lambda i: (i, 0)),
          pl.BlockSpec((D, D), lambda i: (0, 0)),
      ],
      out_specs=pl.BlockSpec((_BLK, D), lambda i: (i, 0)),
      out_shape=jax.ShapeDtypeStruct((N_NODES, D), jnp.float32),
  )(x, w)


def _fuse_mm_kernel(p_ref, b_ref, w_ref, o_ref):
  h = jnp.maximum(p_ref[0] + p_ref[1] + b_ref[...], 0.0)
  o_ref[...] = jnp.dot(h, w_ref[...], preferred_element_type=jnp.float32)


def _tc_combine_relu_matmul(p, b, w):
  return pl.pallas_call(
      _fuse_mm_kernel,
      grid=(N_NODES // _BLK,),
      in_specs=[
          pl.BlockSpec((2, _BLK, D), lambda i: (0, i, 0)),
          pl.BlockSpec((1, D), lambda i: (0, 0)),
          pl.BlockSpec((D, D), lambda i: (0, 0)),
      ],
      out_specs=pl.BlockSpec((_BLK, D), lambda i: (i, 0)),
      out_shape=jax.ShapeDtypeStruct((N_NODES, D), jnp.float32),
  )(p, b.reshape(1, D), w)


def _fuse_relu_kernel(p_ref, b_ref, o_ref):
  o_ref[...] = jnp.maximum(p_ref[0] + p_ref[1] + b_ref[...], 0.0)


def _tc_combine_relu(p, b):
  return pl.pallas_call(
      _fuse_relu_kernel,
      grid=(N_NODES // _BLK,),
      in_specs=[
          pl.BlockSpec((2, _BLK, D), lambda i: (0, i, 0)),
          pl.BlockSpec((1, D), lambda i: (0, 0)),
      ],
      out_specs=pl.BlockSpec((_BLK, D), lambda i: (i, 0)),
      out_shape=jax.ShapeDtypeStruct((N_NODES, D), jnp.float32),
  )(p, b.reshape(1, D))


def _prep_edges(edge_index, edge_values):
  row = edge_index[0].astype(jnp.int32)
  col = edge_index[1].astype(jnp.int32)
  val = edge_values.astype(jnp.float32)
  pad = E_PAD - N_EDGES
  row = jnp.pad(row, (0, pad)).reshape(NW, NCHUNK, CHUNK)
  col = jnp.pad(col, (0, pad)).reshape(NW, NCHUNK, CHUNK)
  val = jnp.pad(val, (0, pad))
  val = jnp.broadcast_to(val[:, None], (E_PAD, 16))
  val = val.reshape(NW * NCHUNK, CHUNK * 16)
  return row, col, val


@jax.jit
def _gcn(H, edge_index, edge_values, W0, b0, W1, b1):
  row, col, val = _prep_edges(edge_index, edge_values)
  x0 = _tc_matmul(H, W0)
  p0 = _spmm(x0, col, row, val).reshape(NC, NPAD, D)
  x1 = _tc_combine_relu_matmul(p0, b0, W1)
  p1 = _spmm(x1, col, row, val).reshape(NC, NPAD, D)
  return _tc_combine_relu(p1, b1)


def kernel(H, edge_index, edge_values, W0, b0, W1, b1):
  return _gcn(H, edge_index, edge_values, W0, b0, W1, b1)


# speedup vs baseline: 48.4297x; 1.0088x over previous
import functools

import jax
import jax.numpy as jnp
from jax import lax
from jax.experimental import pallas as pl
from jax.experimental.pallas import tpu as pltpu
from jax.experimental.pallas import tpu_sc as plsc

N_NODES = 10000
D = 128
N_EDGES = 320000

NC = 2
NS = 16
NW = NC * NS
CHUNK = 128
EPT = 10240
NCHUNK = EPT // CHUNK
E_PAD = NW * EPT
NPAD = 10240
ZROWS = NPAD // NS


def _make_spmm():
  mesh = plsc.VectorSubcoreMesh(core_axis_name="c", subcore_axis_name="s")

  @functools.partial(
      pl.kernel,
      mesh=mesh,
      out_type=jax.ShapeDtypeStruct((NC * NPAD, D), jnp.float32),
      scratch_types=[
          pltpu.VMEM((2, CHUNK), jnp.int32),
          pltpu.VMEM((2, CHUNK), jnp.int32),
          pltpu.VMEM((2, CHUNK), jnp.int32),
          pltpu.VMEM((2, CHUNK), jnp.int32),
          pltpu.VMEM((CHUNK * 16,), jnp.float32),
          pltpu.VMEM((CHUNK * 16,), jnp.float32),
          pltpu.VMEM((CHUNK, D), jnp.float32),
          pltpu.VMEM((CHUNK, D), jnp.float32),
          pltpu.VMEM_SHARED((NPAD, D), jnp.float32),
          pltpu.SemaphoreType.DMA,
          pltpu.SemaphoreType.DMA,
          pltpu.SemaphoreType.DMA,
          pltpu.SemaphoreType.DMA,
          pltpu.SemaphoreType.DMA,
          pltpu.SemaphoreType.DMA,
          pltpu.SemaphoreType.DMA,
          pltpu.SemaphoreType.DMA,
      ],
  )
  def spmm(x_hbm, colrow_hbm, val_hbm, out_hbm,
           cb0, cb1, cb2, cb3, val_v0, val_v1, rows_v0, rows_v1, acc_sh,
           gsem0, gsem1, vsem0, vsem1, csem0, csem1, csem2, csem3):
    cid = lax.axis_index("c")
    sid = lax.axis_index("s")
    tid = cid * NS + sid
    cbuf = (cb0, cb1, cb2, cb3)
    csem = (csem0, csem1, csem2, csem3)
    val_b = (val_v0, val_v1)
    rows_b = (rows_v0, rows_v1)
    gsem_b = (gsem0, gsem1)
    vsem_b = (vsem0, vsem1)

    def colpref(j, q):
      return pltpu.make_async_copy(colrow_hbm.at[tid * NCHUNK + j], cbuf[q],
                                   csem[q])

    def valpref(j, b):
      return pltpu.make_async_copy(val_hbm.at[tid * NCHUNK + j], val_b[b],
                                   vsem_b[b])

    def gather(j, q, b):
      return pltpu.make_async_copy(x_hbm.at[cbuf[q].at[0]], rows_b[b],
                                   gsem_b[b])

    def zero_row(i, carry):
      for f in range(8):
        rows_v0[i, pl.ds(f * 16, 16)] = jnp.zeros((16,), jnp.float32)
      return carry
    lax.fori_loop(0, CHUNK, zero_row, 0)
    for k in range(ZROWS // CHUNK):
      pltpu.sync_copy(rows_v0, acc_sh.at[pl.ds(sid * ZROWS + k * CHUNK, CHUNK)])

    for q in range(4):
      colpref(q, q).start()
    for b in range(2):
      colpref(b, b).wait()
      valpref(b, b).start()
      gather(b, b, b).start()
    plsc.subcore_barrier()

    def chunk_body(i, carry):
      for b in range(4):
        j = 4 * i + b
        rb = b % 2
        valpref(j, rb).wait()
        gather(j, b, rb).wait()

        rv = rows_b[rb]
        vv = val_b[rb]

        @plsc.parallel_loop(0, CHUNK, 1, unroll=4)
        def edge_body(e):
          vb = vv[pl.ds(e * 16, 16)]
          for f in range(8):
            sl = pl.ds(f * 16, 16)
            rv[e, sl] = rv[e, sl] * vb

        pltpu.sync_copy(rv, acc_sh.at[cbuf[b].at[1]], add=True)

        @pl.when(j + 2 < NCHUNK)
        def _():
          colpref(j + 2, (b + 2) % 4).wait()
          valpref(j + 2, rb).start()
          gather(j + 2, (b + 2) % 4, rb).start()

        @pl.when(j + 4 < NCHUNK)
        def _():
          colpref(j + 4, b).start()
      return carry
    lax.fori_loop(0, NCHUNK // 4, chunk_body, 0)

    plsc.subcore_barrier()
    pltpu.sync_copy(acc_sh.at[pl.ds(sid * ZROWS, ZROWS)],
                    out_hbm.at[pl.ds(cid * NPAD + sid * ZROWS, ZROWS)])

  return spmm


_spmm = _make_spmm()


_BLK = 1000


def _mm_kernel(x_ref, w_ref, o_ref):
  o_ref[...] = jnp.dot(x_ref[...], w_ref[...],
                       preferred_element_type=jnp.float32)


def _tc_matmul(x, w):
  return pl.pallas_call(
      _mm_kernel,
      grid=(N_NODES // _BLK,),
      in_specs=[
          pl.BlockSpec((_BLK, D), lambda i: (i, 0)),
          pl.BlockSpec((D, D), lambda i: (0, 0)),
      ],
      out_specs=pl.BlockSpec((_BLK, D), lambda i: (i, 0)),
      out_shape=jax.ShapeDtypeStruct((N_NODES, D), jnp.float32),
  )(x, w)


def _fuse_mm_kernel(p_ref, b_ref, w_ref, o_ref):
  h = jnp.maximum(p_ref[0] + p_ref[1] + b_ref[...], 0.0)
  o_ref[...] = jnp.dot(h, w_ref[...], preferred_element_type=jnp.float32)


def _tc_combine_relu_matmul(p, b, w):
  return pl.pallas_call(
      _fuse_mm_kernel,
      grid=(N_NODES // _BLK,),
      in_specs=[
          pl.BlockSpec((2, _BLK, D), lambda i: (0, i, 0)),
          pl.BlockSpec((1, D), lambda i: (0, 0)),
          pl.BlockSpec((D, D), lambda i: (0, 0)),
      ],
      out_specs=pl.BlockSpec((_BLK, D), lambda i: (i, 0)),
      out_shape=jax.ShapeDtypeStruct((N_NODES, D), jnp.float32),
  )(p, b.reshape(1, D), w)


def _fuse_relu_kernel(p_ref, b_ref, o_ref):
  o_ref[...] = jnp.maximum(p_ref[0] + p_ref[1] + b_ref[...], 0.0)


def _tc_combine_relu(p, b):
  return pl.pallas_call(
      _fuse_relu_kernel,
      grid=(N_NODES // _BLK,),
      in_specs=[
          pl.BlockSpec((2, _BLK, D), lambda i: (0, i, 0)),
          pl.BlockSpec((1, D), lambda i: (0, 0)),
      ],
      out_specs=pl.BlockSpec((_BLK, D), lambda i: (i, 0)),
      out_shape=jax.ShapeDtypeStruct((N_NODES, D), jnp.float32),
  )(p, b.reshape(1, D))


def _prep_edges(edge_index, edge_values):
  row = edge_index[0].astype(jnp.int32)
  col = edge_index[1].astype(jnp.int32)
  val = edge_values.astype(jnp.float32)
  pad = E_PAD - N_EDGES
  row = jnp.pad(row, (0, pad)).reshape(NW * NCHUNK, 1, CHUNK)
  col = jnp.pad(col, (0, pad)).reshape(NW * NCHUNK, 1, CHUNK)
  colrow = jnp.concatenate([col, row], axis=1)
  val = jnp.pad(val, (0, pad))
  val = jnp.broadcast_to(val[:, None], (E_PAD, 16))
  val = val.reshape(NW * NCHUNK, CHUNK * 16)
  return colrow, val


@jax.jit
def _gcn(H, edge_index, edge_values, W0, b0, W1, b1):
  colrow, val = _prep_edges(edge_index, edge_values)
  x0 = _tc_matmul(H, W0)
  p0 = _spmm(x0, colrow, val).reshape(NC, NPAD, D)
  x1 = _tc_combine_relu_matmul(p0, b0, W1)
  p1 = _spmm(x1, colrow, val).reshape(NC, NPAD, D)
  return _tc_combine_relu(p1, b1)


def kernel(H, edge_index, edge_values, W0, b0, W1, b1):
  return _gcn(H, edge_index, edge_values, W0, b0, W1, b1)


# speedup vs baseline: 62.0474x; 1.2812x over previous
import functools

import jax
import jax.numpy as jnp
from jax import lax
from jax.experimental import pallas as pl
from jax.experimental.pallas import tpu as pltpu
from jax.experimental.pallas import tpu_sc as plsc

N_NODES = 10000
D = 128
N_EDGES = 320000

NC = 2
NS = 16
CHUNK = 32
EPT = 20480
NCHUNK = EPT // CHUNK
E_PAD = NS * EPT
XHALF = N_NODES // NC


def _make_spmm():
  mesh = plsc.VectorSubcoreMesh(core_axis_name="c", subcore_axis_name="s")

  @functools.partial(
      pl.kernel,
      mesh=mesh,
      out_type=jax.ShapeDtypeStruct((NC * N_NODES, D), jnp.float32),
      compiler_params=pltpu.CompilerParams(needs_layout_passes=False),
      scratch_types=[
          pltpu.VMEM((3, CHUNK), jnp.int32),
          pltpu.VMEM((3, CHUNK), jnp.int32),
          pltpu.VMEM((3, CHUNK), jnp.int32),
          pltpu.VMEM((3, CHUNK), jnp.int32),
          pltpu.VMEM((CHUNK, D), jnp.float32),
          pltpu.VMEM((CHUNK, D), jnp.float32),
          pltpu.VMEM_SHARED((XHALF, D), jnp.float32),
          pltpu.VMEM_SHARED((N_NODES, D), jnp.float32),
          pltpu.SemaphoreType.DMA,
          pltpu.SemaphoreType.DMA,
          pltpu.SemaphoreType.DMA,
          pltpu.SemaphoreType.DMA,
          pltpu.SemaphoreType.DMA,
          pltpu.SemaphoreType.DMA,
      ],
  )
  def spmm(x_hbm, colrow_hbm, out_hbm,
           cb0, cb1, cb2, cb3, rows_v0, rows_v1, xs_sh, acc_sh,
           gsem0, gsem1, csem0, csem1, csem2, csem3):
    cid = lax.axis_index("c")
    sid = lax.axis_index("s")
    xbase = cid * XHALF
    cbuf = (cb0, cb1, cb2, cb3)
    csem = (csem0, csem1, csem2, csem3)
    rows_b = (rows_v0, rows_v1)
    gsem_b = (gsem0, gsem1)

    def colpref(j, q):
      return pltpu.make_async_copy(colrow_hbm.at[sid * NCHUNK + j], cbuf[q],
                                   csem[q])

    def gather(j, q, b):
      return pltpu.make_async_copy(xs_sh.at[cbuf[q].at[0]], rows_b[b],
                                   gsem_b[b])

    def munge(q):
      cb = cbuf[q]
      for g in range(CHUNK // 16):
        sl = pl.ds(g * 16, 16)
        cl = cb[0, sl] - xbase
        inb = (cl >= 0) & (cl < XHALF)
        cb[0, sl] = jnp.where(inb, cl, 0)
        cb[2, sl] = jnp.where(inb, cb[2, sl], 0)

    pltpu.sync_copy(x_hbm.at[pl.ds(cid * XHALF + sid * 312, 312)],
                    xs_sh.at[pl.ds(sid * 312, 312)])

    @pl.when(sid == NS - 1)
    def _():
      pltpu.sync_copy(
          x_hbm.at[pl.ds(cid * XHALF + NS * 312, XHALF - NS * 312)],
          xs_sh.at[pl.ds(NS * 312, XHALF - NS * 312)])

    def zero_row(i, carry):
      for f in range(8):
        rows_v0[i, pl.ds(f * 16, 16)] = jnp.zeros((16,), jnp.float32)
      return carry
    lax.fori_loop(0, CHUNK, zero_row, 0)
    for k in range(624 // CHUNK):
      pltpu.sync_copy(rows_v0,
                      acc_sh.at[pl.ds(sid * 624 + k * CHUNK, CHUNK)])
    pltpu.sync_copy(rows_v0.at[pl.ds(0, 16)],
                    acc_sh.at[pl.ds(sid * 624 + 608, 16)])

    @pl.when(sid == NS - 1)
    def _():
      pltpu.sync_copy(rows_v0.at[pl.ds(0, 16)],
                      acc_sh.at[pl.ds(NS * 624, N_NODES - NS * 624)])

    for q in range(4):
      colpref(q, q).start()
    plsc.subcore_barrier()
    for b in range(2):
      colpref(b, b).wait()
      munge(b)
      gather(b, b, b).start()

    def chunk_body(i, carry):
      for b in range(4):
        j = 4 * i + b
        rb = b % 2
        gather(j, b, rb).wait()

        rv = rows_b[rb]
        cb = cbuf[b]

        @plsc.parallel_loop(0, CHUNK, 1, unroll=4)
        def edge_body(e):
          g0 = (e // 16) * 16
          vals16 = plsc.bitcast(cb[2, pl.ds(g0, 16)], jnp.float32)
          vb = lax.gather(
              vals16, jnp.full((16, 1), e - g0, jnp.int32),
              lax.GatherDimensionNumbers(offset_dims=(),
                                         collapsed_slice_dims=(0,),
                                         start_index_map=(0,)),
              slice_sizes=(1,),
              mode=lax.GatherScatterMode.PROMISE_IN_BOUNDS)
          for f in range(8):
            sl = pl.ds(f * 16, 16)
            rv[e, sl] = rv[e, sl] * vb

        pltpu.sync_copy(rv, acc_sh.at[cb.at[1]], add=True)

        @pl.when(j + 2 < NCHUNK)
        def _():
          colpref(j + 2, (b + 2) % 4).wait()
          munge((b + 2) % 4)
          gather(j + 2, (b + 2) % 4, rb).start()

        @pl.when(j + 4 < NCHUNK)
        def _():
          colpref(j + 4, b).start()
      return carry
    lax.fori_loop(0, NCHUNK // 4, chunk_body, 0)

    plsc.subcore_barrier()
    pltpu.sync_copy(acc_sh.at[pl.ds(sid * 624, 624)],
                    out_hbm.at[pl.ds(cid * N_NODES + sid * 624, 624)])

    @pl.when(sid == NS - 1)
    def _():
      pltpu.sync_copy(
          acc_sh.at[pl.ds(NS * 624, N_NODES - NS * 624)],
          out_hbm.at[pl.ds(cid * N_NODES + NS * 624, N_NODES - NS * 624)])

  return spmm


_spmm = _make_spmm()


_BLK = 1000


def _mm_kernel(x_ref, w_ref, o_ref):
  o_ref[...] = jnp.dot(x_ref[...], w_ref[...],
                       preferred_element_type=jnp.float32)


def _tc_matmul(x, w):
  return pl.pallas_call(
      _mm_kernel,
      grid=(N_NODES // _BLK,),
      in_specs=[
          pl.BlockSpec((_BLK, D), lambda i: (i, 0)),
          pl.BlockSpec((D, D), lambda i: (0, 0)),
      ],
      out_specs=pl.BlockSpec((_BLK, D), lambda i: (i, 0)),
      out_shape=jax.ShapeDtypeStruct((N_NODES, D), jnp.float32),
  )(x, w)


def _fuse_mm_kernel(p_ref, b_ref, w_ref, o_ref):
  h = jnp.maximum(p_ref[0] + p_ref[1] + b_ref[...], 0.0)
  o_ref[...] = jnp.dot(h, w_ref[...], preferred_element_type=jnp.float32)


def _tc_combine_relu_matmul(p, b, w):
  return pl.pallas_call(
      _fuse_mm_kernel,
      grid=(N_NODES // _BLK,),
      in_specs=[
          pl.BlockSpec((2, _BLK, D), lambda i: (0, i, 0)),
          pl.BlockSpec((1, D), lambda i: (0, 0)),
          pl.BlockSpec((D, D), lambda i: (0, 0)),
      ],
      out_specs=pl.BlockSpec((_BLK, D), lambda i: (i, 0)),
      out_shape=jax.ShapeDtypeStruct((N_NODES, D), jnp.float32),
  )(p, b.reshape(1, D), w)


def _fuse_relu_kernel(p_ref, b_ref, o_ref):
  o_ref[...] = jnp.maximum(p_ref[0] + p_ref[1] + b_ref[...], 0.0)


def _tc_combine_relu(p, b):
  return pl.pallas_call(
      _fuse_relu_kernel,
      grid=(N_NODES // _BLK,),
      in_specs=[
          pl.BlockSpec((2, _BLK, D), lambda i: (0, i, 0)),
          pl.BlockSpec((1, D), lambda i: (0, 0)),
      ],
      out_specs=pl.BlockSpec((_BLK, D), lambda i: (i, 0)),
      out_shape=jax.ShapeDtypeStruct((N_NODES, D), jnp.float32),
  )(p, b.reshape(1, D))


def _prep_edges(edge_index, edge_values):
  row = edge_index[0].astype(jnp.int32)
  col = edge_index[1].astype(jnp.int32)
  val = edge_values.astype(jnp.float32)
  pad = E_PAD - N_EDGES
  row = jnp.pad(row, (0, pad)).reshape(NS * NCHUNK, 1, CHUNK)
  col = jnp.pad(col, (0, pad)).reshape(NS * NCHUNK, 1, CHUNK)
  vbits = lax.bitcast_convert_type(jnp.pad(val, (0, pad)),
                                   jnp.int32).reshape(NS * NCHUNK, 1, CHUNK)
  colrow = jnp.concatenate([col, row, vbits], axis=1)
  return colrow


@jax.jit
def _gcn(H, edge_index, edge_values, W0, b0, W1, b1):
  colrow = _prep_edges(edge_index, edge_values)
  x0 = _tc_matmul(H, W0)
  p0 = _spmm(x0, colrow).reshape(NC, N_NODES, D)
  x1 = _tc_combine_relu_matmul(p0, b0, W1)
  p1 = _spmm(x1, colrow).reshape(NC, N_NODES, D)
  return _tc_combine_relu(p1, b1)


def kernel(H, edge_index, edge_values, W0, b0, W1, b1):
  return _gcn(H, edge_index, edge_values, W0, b0, W1, b1)
